# Initial kernel scaffold; baseline (speedup 1.0000x reference)
import jax
import jax.numpy as jnp
from jax.experimental import pallas as pl


def kernel(x, edge_index, W1, att_src1, att_dst1, b1, W2, att_src2, att_dst2, b2):
    raise NotImplementedError("write your pallas kernel here")


# speedup vs baseline: 28.2792x; 28.2792x over previous
import functools

import jax
import jax.numpy as jnp
from jax import lax
from jax.experimental import pallas as pl
from jax.experimental.pallas import tpu as pltpu
from jax.experimental.pallas import tpu_sc as plsc

N = 10000
E = 320000
C = 128
H = 2
NEG = 0.2
NS = 16
EPT = E // NS
K = 80
NCH = EPT // K
BLK = 2000
DEN_P = 10112
CH = 64


def _sc_body(src_hbm, dst_hbm, gidx_hbm, feat_hbm, as_hbm, ad_hbm,
             zacc_hbm, zden_hbm,
             acc_out, den_out,
             src2d, dst2d, g2d, as_buf, ad_buf, rows, sem, acc_sh, den_sh):
    cid = lax.axis_index("c")
    sid = lax.axis_index("s")

    pltpu.sync_copy(src_hbm.at[sid], src2d)
    pltpu.sync_copy(dst_hbm.at[sid], dst2d)
    pltpu.sync_copy(as_hbm.at[cid], as_buf)
    pltpu.sync_copy(ad_hbm.at[cid], ad_buf)

    @pl.when(sid == 0)
    def _zero():
        pltpu.sync_copy(zacc_hbm, acc_sh)
        pltpu.sync_copy(zden_hbm, den_sh)

    def phase_a(c, carry):
        for half in range(K // 16):
            s = src2d[c, pl.ds(half * 16, 16)]
            d = dst2d[c, pl.ds(half * 16, 16)]
            av = plsc.load_gather(as_buf, [s])
            bv = plsc.load_gather(ad_buf, [d])
            e = av + bv
            e = jnp.where(e >= 0, e, e * NEG)
            g2d[c, pl.ds(half * 16, 16)] = jnp.exp(e)
        return carry

    lax.fori_loop(0, NCH, phase_a, 0)

    plsc.subcore_barrier()

    def make_pass(p):
        def phase_b(c, carry):
            pltpu.async_copy(feat_hbm.at[src2d.at[c]], rows, sem).wait()
            cv = jnp.full((16,), c, jnp.int32)
            for j in range(K):
                jv = jnp.full((16,), j, jnp.int32)
                w = plsc.load_gather(g2d, [cv, jv])
                for q in range(CH // 16):
                    rows[j, pl.ds(q * 16, 16)] = rows[j, pl.ds(q * 16, 16)] * w
            pltpu.sync_copy(rows, acc_sh.at[dst2d.at[c]], add=True)
            if p == 0:
                pltpu.sync_copy(g2d.at[c], den_sh.at[dst2d.at[c]], add=True)
            return carry
        return phase_b

    pltpu.sync_copy(gidx_hbm.at[cid, 0, sid], src2d)
    lax.fori_loop(0, NCH, make_pass(0), 0)
    plsc.subcore_barrier()

    @pl.when(sid == 0)
    def _wb0():
        pltpu.sync_copy(acc_sh, acc_out.at[cid, 0])
        pltpu.sync_copy(zacc_hbm, acc_sh)

    plsc.subcore_barrier()
    pltpu.sync_copy(gidx_hbm.at[cid, 1, sid], src2d)
    lax.fori_loop(0, NCH, make_pass(1), 0)
    plsc.subcore_barrier()

    @pl.when(sid == 0)
    def _wb1():
        pltpu.sync_copy(acc_sh, acc_out.at[cid, 1])
        pltpu.sync_copy(den_sh, den_out.at[cid])


@functools.cache
def _sc_gat_kernel():
    return pl.kernel(
        _sc_body,
        mesh=plsc.VectorSubcoreMesh(core_axis_name="c", subcore_axis_name="s"),
        compiler_params=pltpu.CompilerParams(needs_layout_passes=False,
                                             use_tc_tiling_on_sc=False),
        out_type=[jax.ShapeDtypeStruct((H, 2, N, CH), jnp.float32),
                  jax.ShapeDtypeStruct((H, DEN_P), jnp.float32)],
        scratch_types=[
            pltpu.VMEM((NCH, K), jnp.int32),
            pltpu.VMEM((NCH, K), jnp.int32),
            pltpu.VMEM((NCH, K), jnp.float32),
            pltpu.VMEM((N,), jnp.float32),
            pltpu.VMEM((N,), jnp.float32),
            pltpu.VMEM((K, CH), jnp.float32),
            pltpu.SemaphoreType.DMA,
            pltpu.VMEM_SHARED((N, CH), jnp.float32),
            pltpu.VMEM_SHARED((DEN_P,), jnp.float32),
        ],
    )


def _sc_gat(*args):
    return _sc_gat_kernel()(*args)


def _mm_body(x_ref, w_ref, as_ref, ad_ref, h_ref, av_ref):
    xb = x_ref[...]
    h = jnp.dot(xb, w_ref[...], preferred_element_type=jnp.float32)
    h0 = h[:, :C]
    h1 = h[:, C:]
    h_ref[0] = h0
    h_ref[1] = h1
    a = as_ref[...]
    b = ad_ref[...]
    av_ref[...] = jnp.concatenate([
        jnp.sum(h0 * a[0:1], axis=1, keepdims=True),
        jnp.sum(h1 * a[1:2], axis=1, keepdims=True),
        jnp.sum(h0 * b[0:1], axis=1, keepdims=True),
        jnp.sum(h1 * b[1:2], axis=1, keepdims=True),
    ], axis=1)


def _mm(x, W, att_s, att_d):
    return pl.pallas_call(
        _mm_body,
        grid=(N // BLK,),
        in_specs=[pl.BlockSpec((BLK, C), lambda i: (i, 0)),
                  pl.BlockSpec((C, H * C), lambda i: (0, 0)),
                  pl.BlockSpec((H, C), lambda i: (0, 0)),
                  pl.BlockSpec((H, C), lambda i: (0, 0))],
        out_specs=[pl.BlockSpec((H, BLK, C), lambda i: (0, i, 0)),
                   pl.BlockSpec((BLK, 4), lambda i: (i, 0))],
        out_shape=[jax.ShapeDtypeStruct((H, N, C), jnp.float32),
                   jax.ShapeDtypeStruct((N, 4), jnp.float32)],
    )(x, W, att_s, att_d)


def _comb_body(acc_ref, den_ref, b_ref, o_ref):
    d = den_ref[...] + 1e-16
    a = acc_ref[...]
    o_ref[...] = (a[0] / d[:, 0:1] + a[1] / d[:, 1:2]) * 0.5 + b_ref[...]


def _comb(acc, den, br):
    return pl.pallas_call(
        _comb_body,
        grid=(N // BLK,),
        in_specs=[pl.BlockSpec((H, BLK, C), lambda i: (0, i, 0)),
                  pl.BlockSpec((BLK, H), lambda i: (i, 0)),
                  pl.BlockSpec((1, C), lambda i: (0, 0))],
        out_specs=pl.BlockSpec((BLK, C), lambda i: (i, 0)),
        out_shape=jax.ShapeDtypeStruct((N, C), jnp.float32),
    )(acc, den, br)


def kernel(x, edge_index, W1, att_src1, att_dst1, b1, W2, att_src2, att_dst2, b2):
    ei = edge_index.astype(jnp.int32)
    src3 = ei[0].reshape(NS, NCH, K)
    dst3 = ei[1].reshape(NS, NCH, K)
    hd = jnp.arange(H, dtype=jnp.int32)[:, None, None, None, None]
    pp = jnp.arange(2, dtype=jnp.int32)[None, :, None, None, None]
    gidx = (hd * N + src3[None, None]) * 2 + pp
    zacc = jnp.zeros((N, CH), jnp.float32)
    zden = jnp.zeros((DEN_P,), jnp.float32)

    def layer(xin, W, a_s, a_d, bias):
        h, av = _mm(xin, W, a_s, a_d)
        avt = av.T
        feat = h.reshape(H * N * 2, CH)
        acc, den = _sc_gat(src3, dst3, gidx, feat, avt[0:2], avt[2:4],
                           zacc, zden)
        accf = jnp.transpose(acc, (0, 2, 1, 3)).reshape(H, N, C)
        return _comb(accf, den[:, :N].T, bias.reshape(1, C))

    h1 = layer(x, W1, att_src1, att_dst1, b1)
    return layer(h1, W2, att_src2, att_dst2, b2)


# speedup vs baseline: 35.1063x; 1.2414x over previous
import functools

import jax
import jax.numpy as jnp
from jax import lax
from jax.experimental import pallas as pl
from jax.experimental.pallas import tpu as pltpu
from jax.experimental.pallas import tpu_sc as plsc

N = 10000
E = 320000
C = 128
H = 2
NEG = 0.2
NS = 16
EPT = E // NS
K = 80
NCH = EPT // K
BLK = 2000
DEN_P = 10112
CH = 64


def _sc_body(src_hbm, dst_hbm, gidx_hbm, feat_hbm, as_hbm, ad_hbm,
             zacc_hbm, zden_hbm,
             acc_out, den_out,
             src2d, dst2d, g2d, as_buf, ad_buf, rows0, rows1,
             gsem0, gsem1, ssem0, ssem1, acc_sh, den_sh):
    cid = lax.axis_index("c")
    sid = lax.axis_index("s")

    pltpu.sync_copy(src_hbm.at[sid], src2d)
    pltpu.sync_copy(dst_hbm.at[sid], dst2d)
    pltpu.sync_copy(as_hbm.at[cid], as_buf)
    pltpu.sync_copy(ad_hbm.at[cid], ad_buf)

    @pl.when(sid == 0)
    def _zero():
        pltpu.sync_copy(zacc_hbm, acc_sh)
        pltpu.sync_copy(zden_hbm, den_sh)

    def phase_a(c, carry):
        for half in range(K // 16):
            s = src2d[c, pl.ds(half * 16, 16)]
            d = dst2d[c, pl.ds(half * 16, 16)]
            av = plsc.load_gather(as_buf, [s])
            bv = plsc.load_gather(ad_buf, [d])
            e = av + bv
            e = jnp.where(e >= 0, e, e * NEG)
            g2d[c, pl.ds(half * 16, 16)] = jnp.exp(e)
        return carry

    lax.fori_loop(0, NCH, phase_a, 0)

    plsc.subcore_barrier()

    ROWS = (rows0, rows1)
    GSEM = (gsem0, gsem1)
    SSEM = (ssem0, ssem1)

    def _gather(c, b):
        return pltpu.make_async_copy(feat_hbm.at[src2d.at[c]], ROWS[b], GSEM[b])

    def _scat_drain(b):
        pltpu.make_async_copy(zacc_hbm.at[pl.ds(0, K)], ROWS[b], SSEM[b]).wait()

    def make_pass(p):
        def phase_b(i, carry):
            for b in range(2):
                c = 2 * i + b
                rows = ROWS[b]
                _gather(c, b).wait()
                if b == 0:
                    @pl.when(i > 0)
                    def _dr1():
                        _scat_drain(1)
                    _gather(c + 1, 1).start()
                else:
                    @pl.when(i < NCH // 2 - 1)
                    def _rf0():
                        _scat_drain(0)
                        _gather(c + 1, 0).start()

                cv = jnp.full((16,), c, jnp.int32)
                for j in range(K):
                    jv = jnp.full((16,), j, jnp.int32)
                    w = plsc.load_gather(g2d, [cv, jv])
                    for q in range(CH // 16):
                        rows[j, pl.ds(q * 16, 16)] = rows[j, pl.ds(q * 16, 16)] * w
                pltpu.make_async_copy(rows, acc_sh.at[dst2d.at[c]],
                                      SSEM[b]).start(add=True)
                if p == 0:
                    pltpu.sync_copy(g2d.at[c], den_sh.at[dst2d.at[c]], add=True)
            return carry
        return phase_b

    def run_pass(p):
        pltpu.sync_copy(gidx_hbm.at[cid, p, sid], src2d)
        _gather(0, 0).start()
        lax.fori_loop(0, NCH // 2, make_pass(p), 0)
        _scat_drain(0)
        _scat_drain(1)

    run_pass(0)
    plsc.subcore_barrier()

    @pl.when(sid == 0)
    def _wb0():
        pltpu.sync_copy(acc_sh, acc_out.at[cid, 0])
        pltpu.sync_copy(zacc_hbm, acc_sh)

    plsc.subcore_barrier()
    run_pass(1)
    plsc.subcore_barrier()

    @pl.when(sid == 0)
    def _wb1():
        pltpu.sync_copy(acc_sh, acc_out.at[cid, 1])
        pltpu.sync_copy(den_sh, den_out.at[cid])


@functools.cache
def _sc_gat_kernel():
    return pl.kernel(
        _sc_body,
        mesh=plsc.VectorSubcoreMesh(core_axis_name="c", subcore_axis_name="s"),
        compiler_params=pltpu.CompilerParams(needs_layout_passes=False,
                                             use_tc_tiling_on_sc=False),
        out_type=[jax.ShapeDtypeStruct((H, 2, N, CH), jnp.float32),
                  jax.ShapeDtypeStruct((H, DEN_P), jnp.float32)],
        scratch_types=[
            pltpu.VMEM((NCH, K), jnp.int32),
            pltpu.VMEM((NCH, K), jnp.int32),
            pltpu.VMEM((NCH, K), jnp.float32),
            pltpu.VMEM((N,), jnp.float32),
            pltpu.VMEM((N,), jnp.float32),
            pltpu.VMEM((K, CH), jnp.float32),
            pltpu.VMEM((K, CH), jnp.float32),
            pltpu.SemaphoreType.DMA,
            pltpu.SemaphoreType.DMA,
            pltpu.SemaphoreType.DMA,
            pltpu.SemaphoreType.DMA,
            pltpu.VMEM_SHARED((N, CH), jnp.float32),
            pltpu.VMEM_SHARED((DEN_P,), jnp.float32),
        ],
    )


def _sc_gat(*args):
    return _sc_gat_kernel()(*args)


def _mm_body(x_ref, w_ref, as_ref, ad_ref, h_ref, av_ref):
    xb = x_ref[...]
    h = jnp.dot(xb, w_ref[...], preferred_element_type=jnp.float32)
    h0 = h[:, :C]
    h1 = h[:, C:]
    h_ref[0] = h0
    h_ref[1] = h1
    a = as_ref[...]
    b = ad_ref[...]
    av_ref[...] = jnp.concatenate([
        jnp.sum(h0 * a[0:1], axis=1, keepdims=True),
        jnp.sum(h1 * a[1:2], axis=1, keepdims=True),
        jnp.sum(h0 * b[0:1], axis=1, keepdims=True),
        jnp.sum(h1 * b[1:2], axis=1, keepdims=True),
    ], axis=1)


def _mm(x, W, att_s, att_d):
    return pl.pallas_call(
        _mm_body,
        grid=(N // BLK,),
        in_specs=[pl.BlockSpec((BLK, C), lambda i: (i, 0)),
                  pl.BlockSpec((C, H * C), lambda i: (0, 0)),
                  pl.BlockSpec((H, C), lambda i: (0, 0)),
                  pl.BlockSpec((H, C), lambda i: (0, 0))],
        out_specs=[pl.BlockSpec((H, BLK, C), lambda i: (0, i, 0)),
                   pl.BlockSpec((BLK, 4), lambda i: (i, 0))],
        out_shape=[jax.ShapeDtypeStruct((H, N, C), jnp.float32),
                   jax.ShapeDtypeStruct((N, 4), jnp.float32)],
    )(x, W, att_s, att_d)


def _comb_body(acc_ref, den_ref, b_ref, o_ref):
    d = den_ref[...] + 1e-16
    a = acc_ref[...]
    o_ref[...] = (a[0] / d[:, 0:1] + a[1] / d[:, 1:2]) * 0.5 + b_ref[...]


def _comb(acc, den, br):
    return pl.pallas_call(
        _comb_body,
        grid=(N // BLK,),
        in_specs=[pl.BlockSpec((H, BLK, C), lambda i: (0, i, 0)),
                  pl.BlockSpec((BLK, H), lambda i: (i, 0)),
                  pl.BlockSpec((1, C), lambda i: (0, 0))],
        out_specs=pl.BlockSpec((BLK, C), lambda i: (i, 0)),
        out_shape=jax.ShapeDtypeStruct((N, C), jnp.float32),
    )(acc, den, br)


def kernel(x, edge_index, W1, att_src1, att_dst1, b1, W2, att_src2, att_dst2, b2):
    ei = edge_index.astype(jnp.int32)
    src3 = ei[0].reshape(NS, NCH, K)
    dst3 = ei[1].reshape(NS, NCH, K)
    hd = jnp.arange(H, dtype=jnp.int32)[:, None, None, None, None]
    pp = jnp.arange(2, dtype=jnp.int32)[None, :, None, None, None]
    gidx = (hd * N + src3[None, None]) * 2 + pp
    zacc = jnp.zeros((N, CH), jnp.float32)
    zden = jnp.zeros((DEN_P,), jnp.float32)

    def layer(xin, W, a_s, a_d, bias):
        h, av = _mm(xin, W, a_s, a_d)
        avt = av.T
        feat = h.reshape(H * N * 2, CH)
        acc, den = _sc_gat(src3, dst3, gidx, feat, avt[0:2], avt[2:4],
                           zacc, zden)
        accf = jnp.transpose(acc, (0, 2, 1, 3)).reshape(H, N, C)
        return _comb(accf, den[:, :N].T, bias.reshape(1, C))

    h1 = layer(x, W1, att_src1, att_dst1, b1)
    return layer(h1, W2, att_src2, att_dst2, b2)


# speedup vs baseline: 35.9142x; 1.0230x over previous
import functools

import jax
import jax.numpy as jnp
from jax import lax
from jax.experimental import pallas as pl
from jax.experimental.pallas import tpu as pltpu
from jax.experimental.pallas import tpu_sc as plsc

N = 10000
E = 320000
C = 128
H = 2
NEG = 0.2
NS = 16
EPT = E // NS
K = 80
NCH = EPT // K
BLK = 2000
DEN_P = 10112
CH = 64


def _sc_body(src_hbm, dst_hbm, gidx_hbm, feat_hbm, as_hbm, ad_hbm,
             zacc_hbm, zden_hbm,
             acc_out, den_out,
             src2d, dst2d, g2d, as_buf, ad_buf, rows0, rows1,
             gsem0, gsem1, ssem0, ssem1, densem0, densem1, acc_sh, den_sh):
    cid = lax.axis_index("c")
    sid = lax.axis_index("s")

    pltpu.sync_copy(src_hbm.at[sid], src2d)
    pltpu.sync_copy(dst_hbm.at[sid], dst2d)
    pltpu.sync_copy(as_hbm.at[cid], as_buf)
    pltpu.sync_copy(ad_hbm.at[cid], ad_buf)

    @pl.when(sid == 0)
    def _zero():
        pltpu.sync_copy(zacc_hbm, acc_sh)
        pltpu.sync_copy(zden_hbm, den_sh)

    def phase_a(c, carry):
        for half in range(K // 16):
            s = src2d[c, pl.ds(half * 16, 16)]
            d = dst2d[c, pl.ds(half * 16, 16)]
            av = plsc.load_gather(as_buf, [s])
            bv = plsc.load_gather(ad_buf, [d])
            e = av + bv
            e = jnp.where(e >= 0, e, e * NEG)
            g2d[c, pl.ds(half * 16, 16)] = jnp.exp(e)
        return carry

    lax.fori_loop(0, NCH, phase_a, 0)

    plsc.subcore_barrier()

    ROWS = (rows0, rows1)
    GSEM = (gsem0, gsem1)
    SSEM = (ssem0, ssem1)

    def _gather(c, b):
        return pltpu.make_async_copy(feat_hbm.at[src2d.at[c]], ROWS[b], GSEM[b])

    def _scat_drain(b):
        pltpu.make_async_copy(zacc_hbm.at[pl.ds(0, K)], ROWS[b], SSEM[b]).wait()

    DSEM = (densem0, densem1)

    def _den_drain(b):
        pltpu.make_async_copy(zden_hbm.at[pl.ds(0, K)], g2d.at[0],
                              DSEM[b]).wait()

    def make_pass(p):
        def phase_b(i, carry):
            for b in range(2):
                c = 2 * i + b
                rows = ROWS[b]
                _gather(c, b).wait()
                if b == 0:
                    @pl.when(i > 0)
                    def _dr1():
                        _scat_drain(1)
                    _gather(c + 1, 1).start()
                else:
                    @pl.when(i < NCH // 2 - 1)
                    def _rf0():
                        _scat_drain(0)
                        _gather(c + 1, 0).start()

                if p == 0:
                    @pl.when(i > 0)
                    def _drd():
                        _den_drain(b)

                cv = jnp.full((16,), c, jnp.int32)
                for j in range(K):
                    jv = jnp.full((16,), j, jnp.int32)
                    w = plsc.load_gather(g2d, [cv, jv])
                    for q in range(CH // 16):
                        rows[j, pl.ds(q * 16, 16)] = rows[j, pl.ds(q * 16, 16)] * w
                pltpu.make_async_copy(rows, acc_sh.at[dst2d.at[c]],
                                      SSEM[b]).start(add=True)
                if p == 0:
                    pltpu.make_async_copy(g2d.at[c], den_sh.at[dst2d.at[c]],
                                          DSEM[b]).start(add=True)
            return carry
        return phase_b

    def run_pass(p):
        pltpu.sync_copy(gidx_hbm.at[cid, p, sid], src2d)
        _gather(0, 0).start()
        lax.fori_loop(0, NCH // 2, make_pass(p), 0)
        _scat_drain(0)
        _scat_drain(1)
        if p == 0:
            _den_drain(0)
            _den_drain(1)

    run_pass(0)
    plsc.subcore_barrier()

    @pl.when(sid == 0)
    def _wb0():
        pltpu.sync_copy(acc_sh, acc_out.at[cid, 0])
        pltpu.sync_copy(zacc_hbm, acc_sh)

    plsc.subcore_barrier()
    run_pass(1)
    plsc.subcore_barrier()

    @pl.when(sid == 0)
    def _wb1():
        pltpu.sync_copy(acc_sh, acc_out.at[cid, 1])
        pltpu.sync_copy(den_sh, den_out.at[cid])


@functools.cache
def _sc_gat_kernel():
    return pl.kernel(
        _sc_body,
        mesh=plsc.VectorSubcoreMesh(core_axis_name="c", subcore_axis_name="s"),
        compiler_params=pltpu.CompilerParams(needs_layout_passes=False,
                                             use_tc_tiling_on_sc=False),
        out_type=[jax.ShapeDtypeStruct((H, 2, N, CH), jnp.float32),
                  jax.ShapeDtypeStruct((H, DEN_P), jnp.float32)],
        scratch_types=[
            pltpu.VMEM((NCH, K), jnp.int32),
            pltpu.VMEM((NCH, K), jnp.int32),
            pltpu.VMEM((NCH, K), jnp.float32),
            pltpu.VMEM((N,), jnp.float32),
            pltpu.VMEM((N,), jnp.float32),
            pltpu.VMEM((K, CH), jnp.float32),
            pltpu.VMEM((K, CH), jnp.float32),
            pltpu.SemaphoreType.DMA,
            pltpu.SemaphoreType.DMA,
            pltpu.SemaphoreType.DMA,
            pltpu.SemaphoreType.DMA,
            pltpu.SemaphoreType.DMA,
            pltpu.SemaphoreType.DMA,
            pltpu.VMEM_SHARED((N, CH), jnp.float32),
            pltpu.VMEM_SHARED((DEN_P,), jnp.float32),
        ],
    )


def _sc_gat(*args):
    return _sc_gat_kernel()(*args)


def _mm_body(x_ref, w_ref, as_ref, ad_ref, h_ref, av_ref):
    xb = x_ref[...]
    h = jnp.dot(xb, w_ref[...], preferred_element_type=jnp.float32)
    h0 = h[:, :C]
    h1 = h[:, C:]
    h_ref[0] = h0
    h_ref[1] = h1
    a = as_ref[...]
    b = ad_ref[...]
    av_ref[...] = jnp.concatenate([
        jnp.sum(h0 * a[0:1], axis=1, keepdims=True),
        jnp.sum(h1 * a[1:2], axis=1, keepdims=True),
        jnp.sum(h0 * b[0:1], axis=1, keepdims=True),
        jnp.sum(h1 * b[1:2], axis=1, keepdims=True),
    ], axis=1)


def _mm(x, W, att_s, att_d):
    return pl.pallas_call(
        _mm_body,
        grid=(N // BLK,),
        in_specs=[pl.BlockSpec((BLK, C), lambda i: (i, 0)),
                  pl.BlockSpec((C, H * C), lambda i: (0, 0)),
                  pl.BlockSpec((H, C), lambda i: (0, 0)),
                  pl.BlockSpec((H, C), lambda i: (0, 0))],
        out_specs=[pl.BlockSpec((H, BLK, C), lambda i: (0, i, 0)),
                   pl.BlockSpec((BLK, 4), lambda i: (i, 0))],
        out_shape=[jax.ShapeDtypeStruct((H, N, C), jnp.float32),
                   jax.ShapeDtypeStruct((N, 4), jnp.float32)],
    )(x, W, att_s, att_d)


def _comb_body(acc_ref, den_ref, b_ref, o_ref):
    d = den_ref[...] + 1e-16
    a = acc_ref[...]
    o_ref[...] = (a[0] / d[:, 0:1] + a[1] / d[:, 1:2]) * 0.5 + b_ref[...]


def _comb(acc, den, br):
    return pl.pallas_call(
        _comb_body,
        grid=(N // BLK,),
        in_specs=[pl.BlockSpec((H, BLK, C), lambda i: (0, i, 0)),
                  pl.BlockSpec((BLK, H), lambda i: (i, 0)),
                  pl.BlockSpec((1, C), lambda i: (0, 0))],
        out_specs=pl.BlockSpec((BLK, C), lambda i: (i, 0)),
        out_shape=jax.ShapeDtypeStruct((N, C), jnp.float32),
    )(acc, den, br)


def kernel(x, edge_index, W1, att_src1, att_dst1, b1, W2, att_src2, att_dst2, b2):
    ei = edge_index.astype(jnp.int32)
    src3 = ei[0].reshape(NS, NCH, K)
    dst3 = ei[1].reshape(NS, NCH, K)
    hd = jnp.arange(H, dtype=jnp.int32)[:, None, None, None, None]
    pp = jnp.arange(2, dtype=jnp.int32)[None, :, None, None, None]
    gidx = (hd * N + src3[None, None]) * 2 + pp
    zacc = jnp.zeros((N, CH), jnp.float32)
    zden = jnp.zeros((DEN_P,), jnp.float32)

    def layer(xin, W, a_s, a_d, bias):
        h, av = _mm(xin, W, a_s, a_d)
        avt = av.T
        feat = h.reshape(H * N * 2, CH)
        acc, den = _sc_gat(src3, dst3, gidx, feat, avt[0:2], avt[2:4],
                           zacc, zden)
        accf = jnp.transpose(acc, (0, 2, 1, 3)).reshape(H, N, C)
        return _comb(accf, den[:, :N].T, bias.reshape(1, C))

    h1 = layer(x, W1, att_src1, att_dst1, b1)
    return layer(h1, W2, att_src2, att_dst2, b2)
